# initial kernel scaffold (unmeasured)
import jax
import jax.numpy as jnp
from jax import lax
from jax.experimental import pallas as pl
from jax.experimental.pallas import tpu as pltpu

N_DEV = 8
B = 2
SQ = 256
SKV = 256
HQ_LOC = 4
DH = 64
D_MODEL = 512
BLK = 64
BFLY_BITS = (1, 3, 4)


def kernel(x, Wq, K_ext, V_ext, Wo):
    def body(x_ref, wq_ref, k_ref, v_ref, wo_ref, out_ref,
             kv_buf, ctx_buf, bfly_buf,
             scat_send_sems, scat_recv_sems, bfly_send_sems, bfly_recv_sems):
        my_pos = lax.axis_index("i")

        def scatter_rdmas():
            rdmas = []
            for p in range(1, N_DEV):
                for idx, src in ((0, k_ref), (1, v_ref)):
                    rdmas.append(pltpu.make_async_remote_copy(
                        src_ref=src.at[:, :, pl.ds(HQ_LOC * p, HQ_LOC), :],
                        dst_ref=kv_buf.at[idx],
                        send_sem=scat_send_sems.at[p - 1, idx],
                        recv_sem=scat_recv_sems.at[idx],
                        device_id=(p,),
                        device_id_type=pl.DeviceIdType.MESH,
                    ))
            return rdmas

        @pl.when(my_pos == 0)
        def _():
            kv_buf[0] = k_ref[:, :, 0:HQ_LOC, :]
            kv_buf[1] = v_ref[:, :, 0:HQ_LOC, :]
            for r in scatter_rdmas():
                r.start()

        @pl.when(my_pos != 0)
        def _():
            for idx in range(2):
                recv = pltpu.make_async_remote_copy(
                    src_ref=kv_buf.at[idx], dst_ref=kv_buf.at[idx],
                    send_sem=scat_send_sems.at[0, idx],
                    recv_sem=scat_recv_sems.at[idx],
                    device_id=(0,), device_id_type=pl.DeviceIdType.MESH,
                )
                recv.wait_recv()

        row = lax.broadcasted_iota(jnp.int32, (SQ, SKV), 0) // BLK
        col = lax.broadcasted_iota(jnp.int32, (SQ, SKV), 1) // BLK
        mask = col <= row
        for b in range(B):
            q_b = jnp.dot(x_ref[b], wq_ref[...],
                          preferred_element_type=jnp.float32)
            for h in range(HQ_LOC):
                q_bh = q_b[:, h * DH:(h + 1) * DH]
                k_bh = kv_buf[0, b, :, h, :]
                v_bh = kv_buf[1, b, :, h, :]
                s = lax.dot_general(
                    q_bh, k_bh, (((1,), (1,)), ((), ())),
                    preferred_element_type=jnp.float32) * 0.125
                s = jnp.where(mask, s, -1e9)
                m = jnp.max(s, axis=1, keepdims=True)
                w = jnp.exp(s - m)
                w = w / jnp.sum(w, axis=1, keepdims=True)
                ctx_buf[b, :, h * DH:(h + 1) * DH] = jnp.dot(
                    w, v_bh, preferred_element_type=jnp.float32)

        for b in range(B):
            out_ref[b] = jnp.dot(ctx_buf[b], wo_ref[...],
                                 preferred_element_type=jnp.float32)

        for s_i, bit in enumerate(BFLY_BITS):
            partner = my_pos ^ bit
            rdma = pltpu.make_async_remote_copy(
                src_ref=out_ref,
                dst_ref=bfly_buf.at[s_i],
                send_sem=bfly_send_sems.at[s_i],
                recv_sem=bfly_recv_sems.at[s_i],
                device_id=(partner,),
                device_id_type=pl.DeviceIdType.MESH,
            )
            rdma.start()
            rdma.wait()
            out_ref[...] = out_ref[...] + bfly_buf[s_i]

        @pl.when(my_pos == 0)
        def _():
            for r in scatter_rdmas():
                r.wait_send()

    return pl.pallas_call(
        body,
        out_shape=jax.ShapeDtypeStruct((B, SQ, D_MODEL), jnp.float32),
        in_specs=[pl.BlockSpec(memory_space=pltpu.VMEM)] * 5,
        out_specs=pl.BlockSpec(memory_space=pltpu.VMEM),
        scratch_shapes=[
            pltpu.VMEM((2, B, SKV, HQ_LOC, DH), jnp.float32),
            pltpu.VMEM((B, SQ, HQ_LOC * DH), jnp.float32),
            pltpu.VMEM((3, B, SQ, D_MODEL), jnp.float32),
            pltpu.SemaphoreType.DMA((N_DEV - 1, 2)),
            pltpu.SemaphoreType.DMA((2,)),
            pltpu.SemaphoreType.DMA((3,)),
            pltpu.SemaphoreType.DMA((3,)),
        ],
        compiler_params=pltpu.CompilerParams(collective_id=0),
    )(x, Wq, K_ext, V_ext, Wo)


# baseline (device time: 159296 ns/iter reference)
import jax
import jax.numpy as jnp
from jax import lax
from jax.experimental import pallas as pl
from jax.experimental.pallas import tpu as pltpu

N_DEV = 8
B = 2
SQ = 256
SKV = 256
HQ_LOC = 4
DH = 64
D_MODEL = 512
BLK = 64
BFLY_BITS = (1, 3, 4)


def kernel(x, Wq, K_ext, V_ext, Wo):
    def body(x_ref, wq_ref, k_ref, v_ref, wo_ref, out_ref,
             kv_buf, ctx_buf, bfly_buf,
             scat_send_sems, scat_recv_sems, bfly_send_sems, bfly_recv_sems):
        my_pos = lax.axis_index("i")

        def scatter_rdmas():
            rdmas = []
            for p in range(1, N_DEV):
                for idx, src in ((0, k_ref), (1, v_ref)):
                    rdmas.append(pltpu.make_async_remote_copy(
                        src_ref=src.at[:, :, pl.ds(HQ_LOC * p, HQ_LOC), :],
                        dst_ref=kv_buf.at[idx],
                        send_sem=scat_send_sems.at[p - 1, idx],
                        recv_sem=scat_recv_sems.at[idx],
                        device_id=(p,),
                        device_id_type=pl.DeviceIdType.MESH,
                    ))
            return rdmas

        @pl.when(my_pos == 0)
        def _():
            kv_buf[0] = k_ref[:, :, 0:HQ_LOC, :]
            kv_buf[1] = v_ref[:, :, 0:HQ_LOC, :]
            for r in scatter_rdmas():
                r.start()

        @pl.when(my_pos != 0)
        def _():
            for idx in range(2):
                recv = pltpu.make_async_remote_copy(
                    src_ref=kv_buf.at[idx], dst_ref=kv_buf.at[idx],
                    send_sem=scat_send_sems.at[0, idx],
                    recv_sem=scat_recv_sems.at[idx],
                    device_id=(0,), device_id_type=pl.DeviceIdType.MESH,
                )
                recv.wait_recv()

        row = lax.broadcasted_iota(jnp.int32, (SQ, SKV), 0) // BLK
        col = lax.broadcasted_iota(jnp.int32, (SQ, SKV), 1) // BLK
        mask = col <= row
        for b in range(B):
            q_b = jnp.dot(x_ref[b], wq_ref[...],
                          preferred_element_type=jnp.float32)
            for h in range(HQ_LOC):
                q_bh = q_b[:, h * DH:(h + 1) * DH]
                k_bh = kv_buf[0, b, :, h, :]
                v_bh = kv_buf[1, b, :, h, :]
                s = lax.dot_general(
                    q_bh, k_bh, (((1,), (1,)), ((), ())),
                    preferred_element_type=jnp.float32) * 0.125
                s = jnp.where(mask, s, -1e9)
                m = jnp.max(s, axis=1, keepdims=True)
                w = jnp.exp(s - m)
                w = w / jnp.sum(w, axis=1, keepdims=True)
                ctx_buf[b, :, h * DH:(h + 1) * DH] = jnp.dot(
                    w, v_bh, preferred_element_type=jnp.float32)

        for b in range(B):
            out_ref[b] = jnp.dot(ctx_buf[b], wo_ref[...],
                                 preferred_element_type=jnp.float32)

        for s_i, bit in enumerate(BFLY_BITS):
            partner = my_pos ^ bit
            rdma = pltpu.make_async_remote_copy(
                src_ref=out_ref,
                dst_ref=bfly_buf.at[s_i],
                send_sem=bfly_send_sems.at[s_i],
                recv_sem=bfly_recv_sems.at[s_i],
                device_id=(partner,),
                device_id_type=pl.DeviceIdType.MESH,
            )
            rdma.start()
            rdma.wait()
            out_ref[...] = out_ref[...] + bfly_buf[s_i]

        @pl.when(my_pos == 0)
        def _():
            for r in scatter_rdmas():
                r.wait_send()

    return pl.pallas_call(
        body,
        out_shape=jax.ShapeDtypeStruct((B, SQ, D_MODEL), jnp.float32),
        in_specs=[pl.BlockSpec(memory_space=pltpu.VMEM)] * 5,
        out_specs=pl.BlockSpec(memory_space=pltpu.VMEM),
        scratch_shapes=[
            pltpu.VMEM((2, B, SKV, HQ_LOC, DH), jnp.float32),
            pltpu.VMEM((B, SQ, HQ_LOC * DH), jnp.float32),
            pltpu.VMEM((3, B, SQ, D_MODEL), jnp.float32),
            pltpu.SemaphoreType.DMA((N_DEV - 1, 2)),
            pltpu.SemaphoreType.DMA((2,)),
            pltpu.SemaphoreType.DMA((3,)),
            pltpu.SemaphoreType.DMA((3,)),
        ],
    )(x, Wq, K_ext, V_ext, Wo)


# device time: 118344 ns/iter; 1.3460x vs baseline; 1.3460x over previous
import jax
import jax.numpy as jnp
from jax import lax
from jax.experimental import pallas as pl
from jax.experimental.pallas import tpu as pltpu

N_DEV = 8
B = 2
SQ = 256
SKV = 256
HQ_LOC = 4
DH = 64
D_MODEL = 512
BLK = 64
BFLY_BITS = (4, 3, 1)


def kernel(x, Wq, K_ext, V_ext, Wo):
    def body(x_ref, wq_ref, k_ref, v_ref, wo_ref, out_ref,
             kv_buf, stage_ref, ctx_buf, bfly_buf,
             scat_send_sems, scat_recv_sems, bfly_send_sems, bfly_recv_sems):
        my_pos = lax.axis_index("i")

        def scatter_rdmas():
            rdmas = []
            for p in range(1, N_DEV):
                for idx in range(2):
                    rdmas.append(pltpu.make_async_remote_copy(
                        src_ref=stage_ref.at[p - 1, idx],
                        dst_ref=kv_buf.at[idx],
                        send_sem=scat_send_sems.at[p - 1, idx],
                        recv_sem=scat_recv_sems.at[idx],
                        device_id=(p,),
                        device_id_type=pl.DeviceIdType.MESH,
                    ))
            return rdmas

        @pl.when(my_pos == 0)
        def _():
            for p in range(1, N_DEV):
                lo = HQ_LOC * p
                stage_ref[p - 1, 0] = k_ref[:, :, lo:lo + HQ_LOC, :].astype(
                    jnp.bfloat16)
                stage_ref[p - 1, 1] = v_ref[:, :, lo:lo + HQ_LOC, :].astype(
                    jnp.bfloat16)
            for r in scatter_rdmas():
                r.start()
            kv_buf[0] = k_ref[:, :, 0:HQ_LOC, :].astype(jnp.bfloat16)
            kv_buf[1] = v_ref[:, :, 0:HQ_LOC, :].astype(jnp.bfloat16)

        q_all = [jnp.dot(x_ref[b], wq_ref[...],
                         preferred_element_type=jnp.float32) for b in range(B)]

        @pl.when(my_pos != 0)
        def _():
            for idx in range(2):
                recv = pltpu.make_async_remote_copy(
                    src_ref=kv_buf.at[idx], dst_ref=kv_buf.at[idx],
                    send_sem=scat_send_sems.at[0, idx],
                    recv_sem=scat_recv_sems.at[idx],
                    device_id=(0,), device_id_type=pl.DeviceIdType.MESH,
                )
                recv.wait_recv()

        row = lax.broadcasted_iota(jnp.int32, (SQ, SKV), 0) // BLK
        col = lax.broadcasted_iota(jnp.int32, (SQ, SKV), 1) // BLK
        mask = col <= row
        for b in range(B):
            for h in range(HQ_LOC):
                q_bh = q_all[b][:, h * DH:(h + 1) * DH].astype(jnp.bfloat16)
                k_bh = kv_buf[0, b, :, h, :]
                v_bh = kv_buf[1, b, :, h, :]
                s = lax.dot_general(
                    q_bh, k_bh, (((1,), (1,)), ((), ())),
                    preferred_element_type=jnp.float32) * 0.125
                s = jnp.where(mask, s, -1e9)
                m = jnp.max(s, axis=1, keepdims=True)
                w = jnp.exp(s - m)
                w = w / jnp.sum(w, axis=1, keepdims=True)
                ctx_buf[b, :, h * DH:(h + 1) * DH] = jnp.dot(
                    w.astype(jnp.bfloat16), v_bh,
                    preferred_element_type=jnp.float32)

        for b in range(B):
            out_ref[b] = jnp.dot(ctx_buf[b], wo_ref[...],
                                 preferred_element_type=jnp.float32)

        for s_i, bit in enumerate(BFLY_BITS):
            partner = my_pos ^ bit
            rdma = pltpu.make_async_remote_copy(
                src_ref=out_ref,
                dst_ref=bfly_buf.at[s_i],
                send_sem=bfly_send_sems.at[s_i],
                recv_sem=bfly_recv_sems.at[s_i],
                device_id=(partner,),
                device_id_type=pl.DeviceIdType.MESH,
            )
            rdma.start()
            rdma.wait()
            out_ref[...] = out_ref[...] + bfly_buf[s_i]

        @pl.when(my_pos == 0)
        def _():
            for r in scatter_rdmas():
                r.wait_send()

    return pl.pallas_call(
        body,
        out_shape=jax.ShapeDtypeStruct((B, SQ, D_MODEL), jnp.float32),
        in_specs=[pl.BlockSpec(memory_space=pltpu.VMEM)] * 5,
        out_specs=pl.BlockSpec(memory_space=pltpu.VMEM),
        scratch_shapes=[
            pltpu.VMEM((2, B, SKV, HQ_LOC, DH), jnp.bfloat16),
            pltpu.VMEM((N_DEV - 1, 2, B, SKV, HQ_LOC, DH), jnp.bfloat16),
            pltpu.VMEM((B, SQ, HQ_LOC * DH), jnp.float32),
            pltpu.VMEM((3, B, SQ, D_MODEL), jnp.float32),
            pltpu.SemaphoreType.DMA((N_DEV - 1, 2)),
            pltpu.SemaphoreType.DMA((2,)),
            pltpu.SemaphoreType.DMA((3,)),
            pltpu.SemaphoreType.DMA((3,)),
        ],
    )(x, Wq, K_ext, V_ext, Wo)


# device time: 101482 ns/iter; 1.5697x vs baseline; 1.1662x over previous
import jax
import jax.numpy as jnp
from jax import lax
from jax.experimental import pallas as pl
from jax.experimental.pallas import tpu as pltpu

N_DEV = 8
B = 2
SQ = 256
SKV = 256
HQ_LOC = 4
DH = 64
D_MODEL = 512
BLK = 64
BFLY_BITS = (4, 3, 1)


def kernel(x, Wq, K_ext, V_ext, Wo):
    def body(x_ref, wq_ref, k_ref, v_ref, wo_ref, out_ref,
             kv_buf, stage_ref, ctx_buf, bfly_snd, bfly_buf,
             scat_send_sems, scat_recv_sems, bfly_send_sems, bfly_recv_sems):
        my_pos = lax.axis_index("i")

        def scatter_rdmas():
            rdmas = []
            for p in range(1, N_DEV):
                for idx in range(2):
                    rdmas.append(pltpu.make_async_remote_copy(
                        src_ref=stage_ref.at[p - 1, idx],
                        dst_ref=kv_buf.at[idx],
                        send_sem=scat_send_sems.at[p - 1, idx],
                        recv_sem=scat_recv_sems.at[idx],
                        device_id=(p,),
                        device_id_type=pl.DeviceIdType.MESH,
                    ))
            return rdmas

        @pl.when(my_pos == 0)
        def _():
            for p in range(1, N_DEV):
                lo = HQ_LOC * p
                stage_ref[p - 1, 0] = k_ref[:, :, lo:lo + HQ_LOC, :].astype(
                    jnp.bfloat16)
                stage_ref[p - 1, 1] = v_ref[:, :, lo:lo + HQ_LOC, :].astype(
                    jnp.bfloat16)
            for r in scatter_rdmas():
                r.start()
            kv_buf[0] = k_ref[:, :, 0:HQ_LOC, :].astype(jnp.bfloat16)
            kv_buf[1] = v_ref[:, :, 0:HQ_LOC, :].astype(jnp.bfloat16)

        q_all = [jnp.dot(x_ref[b], wq_ref[...],
                         preferred_element_type=jnp.float32) for b in range(B)]

        @pl.when(my_pos != 0)
        def _():
            for idx in range(2):
                recv = pltpu.make_async_remote_copy(
                    src_ref=kv_buf.at[idx], dst_ref=kv_buf.at[idx],
                    send_sem=scat_send_sems.at[0, idx],
                    recv_sem=scat_recv_sems.at[idx],
                    device_id=(0,), device_id_type=pl.DeviceIdType.MESH,
                )
                recv.wait_recv()

        row = lax.broadcasted_iota(jnp.int32, (SQ, SKV), 0) // BLK
        col = lax.broadcasted_iota(jnp.int32, (SQ, SKV), 1) // BLK
        mask = col <= row
        for b in range(B):
            for h in range(HQ_LOC):
                q_bh = q_all[b][:, h * DH:(h + 1) * DH].astype(jnp.bfloat16)
                k_bh = kv_buf[0, b, :, h, :]
                v_bh = kv_buf[1, b, :, h, :]
                s = lax.dot_general(
                    q_bh, k_bh, (((1,), (1,)), ((), ())),
                    preferred_element_type=jnp.float32) * 0.125
                s = jnp.where(mask, s, -1e9)
                m = jnp.max(s, axis=1, keepdims=True)
                w = jnp.exp(s - m)
                w = w / jnp.sum(w, axis=1, keepdims=True)
                ctx_buf[b, :, h * DH:(h + 1) * DH] = jnp.dot(
                    w.astype(jnp.bfloat16), v_bh,
                    preferred_element_type=jnp.float32)

        for b in range(B):
            out_ref[b] = jnp.dot(ctx_buf[b], wo_ref[...],
                                 preferred_element_type=jnp.float32)

        for s_i, bit in enumerate(BFLY_BITS):
            partner = my_pos ^ bit
            bfly_snd[...] = out_ref[...].astype(jnp.bfloat16)
            rdma = pltpu.make_async_remote_copy(
                src_ref=bfly_snd,
                dst_ref=bfly_buf.at[s_i],
                send_sem=bfly_send_sems.at[s_i],
                recv_sem=bfly_recv_sems.at[s_i],
                device_id=(partner,),
                device_id_type=pl.DeviceIdType.MESH,
            )
            rdma.start()
            rdma.wait()
            out_ref[...] = out_ref[...] + bfly_buf[s_i].astype(jnp.float32)

        @pl.when(my_pos == 0)
        def _():
            for r in scatter_rdmas():
                r.wait_send()

    return pl.pallas_call(
        body,
        out_shape=jax.ShapeDtypeStruct((B, SQ, D_MODEL), jnp.float32),
        in_specs=[pl.BlockSpec(memory_space=pltpu.VMEM)] * 5,
        out_specs=pl.BlockSpec(memory_space=pltpu.VMEM),
        scratch_shapes=[
            pltpu.VMEM((2, B, SKV, HQ_LOC, DH), jnp.bfloat16),
            pltpu.VMEM((N_DEV - 1, 2, B, SKV, HQ_LOC, DH), jnp.bfloat16),
            pltpu.VMEM((B, SQ, HQ_LOC * DH), jnp.float32),
            pltpu.VMEM((B, SQ, D_MODEL), jnp.bfloat16),
            pltpu.VMEM((3, B, SQ, D_MODEL), jnp.bfloat16),
            pltpu.SemaphoreType.DMA((N_DEV - 1, 2)),
            pltpu.SemaphoreType.DMA((2,)),
            pltpu.SemaphoreType.DMA((3,)),
            pltpu.SemaphoreType.DMA((3,)),
        ],
    )(x, Wq, K_ext, V_ext, Wo)


# device time: 101467 ns/iter; 1.5699x vs baseline; 1.0001x over previous
import jax
import jax.numpy as jnp
from jax import lax
from jax.experimental import pallas as pl
from jax.experimental.pallas import tpu as pltpu

N_DEV = 8
B = 2
SQ = 256
SKV = 256
HQ_LOC = 4
DH = 64
D_MODEL = 512
BLK = 64
BFLY_BITS = (4, 3, 1)


def kernel(x, Wq, K_ext, V_ext, Wo):
    def body(x_ref, wq_ref, k_ref, v_ref, wo_ref, out_ref,
             kv_buf, stage_ref, ctx_buf, bfly_snd, bfly_buf,
             scat_send_sems, scat_recv_sems, bfly_send_sems, bfly_recv_sems):
        my_pos = lax.axis_index("i")

        def scatter_rdmas():
            rdmas = []
            for p in range(1, N_DEV):
                for idx in range(2):
                    rdmas.append(pltpu.make_async_remote_copy(
                        src_ref=stage_ref.at[p - 1, idx],
                        dst_ref=kv_buf.at[idx],
                        send_sem=scat_send_sems.at[p - 1, idx],
                        recv_sem=scat_recv_sems.at[idx],
                        device_id=(p,),
                        device_id_type=pl.DeviceIdType.MESH,
                    ))
            return rdmas

        with jax.named_scope("stage_and_send"):
            @pl.when(my_pos == 0)
            def _():
                for p in range(1, N_DEV):
                    lo = HQ_LOC * p
                    stage_ref[p - 1, 0] = k_ref[:, :, lo:lo + HQ_LOC, :].astype(
                        jnp.bfloat16)
                    stage_ref[p - 1, 1] = v_ref[:, :, lo:lo + HQ_LOC, :].astype(
                        jnp.bfloat16)
                for r in scatter_rdmas():
                    r.start()
                kv_buf[0] = k_ref[:, :, 0:HQ_LOC, :].astype(jnp.bfloat16)
                kv_buf[1] = v_ref[:, :, 0:HQ_LOC, :].astype(jnp.bfloat16)

        with jax.named_scope("q_proj"):
            q_all = [jnp.dot(x_ref[b], wq_ref[...],
                             preferred_element_type=jnp.float32)
                     for b in range(B)]

        with jax.named_scope("wait_kv"):
            @pl.when(my_pos != 0)
            def _():
                for idx in range(2):
                    recv = pltpu.make_async_remote_copy(
                        src_ref=kv_buf.at[idx], dst_ref=kv_buf.at[idx],
                        send_sem=scat_send_sems.at[0, idx],
                        recv_sem=scat_recv_sems.at[idx],
                        device_id=(0,), device_id_type=pl.DeviceIdType.MESH,
                    )
                    recv.wait_recv()

        with jax.named_scope("attn"):
            row = lax.broadcasted_iota(jnp.int32, (SQ, SKV), 0) // BLK
            col = lax.broadcasted_iota(jnp.int32, (SQ, SKV), 1) // BLK
            mask = col <= row
            for b in range(B):
                for h in range(HQ_LOC):
                    q_bh = q_all[b][:, h * DH:(h + 1) * DH].astype(jnp.bfloat16)
                    k_bh = kv_buf[0, b, :, h, :]
                    v_bh = kv_buf[1, b, :, h, :]
                    s = lax.dot_general(
                        q_bh, k_bh, (((1,), (1,)), ((), ())),
                        preferred_element_type=jnp.float32) * 0.125
                    s = jnp.where(mask, s, -1e9)
                    m = jnp.max(s, axis=1, keepdims=True)
                    w = jnp.exp(s - m)
                    w = w / jnp.sum(w, axis=1, keepdims=True)
                    ctx_buf[b, :, h * DH:(h + 1) * DH] = jnp.dot(
                        w.astype(jnp.bfloat16), v_bh,
                        preferred_element_type=jnp.float32)

        with jax.named_scope("o_proj"):
            for b in range(B):
                out_ref[b] = jnp.dot(ctx_buf[b], wo_ref[...],
                                     preferred_element_type=jnp.float32)

        for s_i, bit in enumerate(BFLY_BITS):
            with jax.named_scope(f"bfly#stage={s_i}"):
                partner = my_pos ^ bit
                bfly_snd[...] = out_ref[...].astype(jnp.bfloat16)
                rdma = pltpu.make_async_remote_copy(
                    src_ref=bfly_snd,
                    dst_ref=bfly_buf.at[s_i],
                    send_sem=bfly_send_sems.at[s_i],
                    recv_sem=bfly_recv_sems.at[s_i],
                    device_id=(partner,),
                    device_id_type=pl.DeviceIdType.MESH,
                )
                rdma.start()
                rdma.wait()
                out_ref[...] = out_ref[...] + bfly_buf[s_i].astype(jnp.float32)

        with jax.named_scope("drain_sends"):
            @pl.when(my_pos == 0)
            def _():
                for r in scatter_rdmas():
                    r.wait_send()

    return pl.pallas_call(
        body,
        out_shape=jax.ShapeDtypeStruct((B, SQ, D_MODEL), jnp.float32),
        in_specs=[pl.BlockSpec(memory_space=pltpu.VMEM)] * 5,
        out_specs=pl.BlockSpec(memory_space=pltpu.VMEM),
        scratch_shapes=[
            pltpu.VMEM((2, B, SKV, HQ_LOC, DH), jnp.bfloat16),
            pltpu.VMEM((N_DEV - 1, 2, B, SKV, HQ_LOC, DH), jnp.bfloat16),
            pltpu.VMEM((B, SQ, HQ_LOC * DH), jnp.float32),
            pltpu.VMEM((B, SQ, D_MODEL), jnp.bfloat16),
            pltpu.VMEM((3, B, SQ, D_MODEL), jnp.bfloat16),
            pltpu.SemaphoreType.DMA((N_DEV - 1, 2)),
            pltpu.SemaphoreType.DMA((2,)),
            pltpu.SemaphoreType.DMA((3,)),
            pltpu.SemaphoreType.DMA((3,)),
        ],
    )(x, Wq, K_ext, V_ext, Wo)
